# Initial kernel scaffold; baseline (speedup 1.0000x reference)
import jax
import jax.numpy as jnp
from jax.experimental import pallas as pl


def kernel(x, edge_index, W1, b1, W2, b2, W3, b3, Wl, bl):
    raise NotImplementedError("write your pallas kernel here")


# speedup vs baseline: 7.3505x; 7.3505x over previous
import jax
import jax.numpy as jnp
from jax import lax
from jax.experimental import pallas as pl
from jax.experimental.pallas import tpu as pltpu
from jax.experimental.pallas import tpu_sc as plsc

N = 10000
E = 320000
F = 128
NCLASS = 40

NC = 2
NS = 16
NW = NC * NS
K = 128
CPW = 80
EPW = CPW * K
EPAD = NW * EPW
SRP = 632
NP = NS * SRP

_MESH = plsc.VectorSubcoreMesh(
    core_axis_name="c", subcore_axis_name="s", num_cores=NC, num_subcores=NS)


def _sc_deg_body(col_hbm, z_hbm, ones_hbm, out_hbm, col_v, ones_v, acc, sem):
    cid = lax.axis_index("c")
    sid = lax.axis_index("s")
    wid = cid * NS + sid
    base = sid * SRP

    pltpu.sync_copy(col_hbm.at[pl.ds(wid * CPW, CPW)], col_v)
    pltpu.sync_copy(ones_hbm, ones_v)
    pltpu.sync_copy(z_hbm.at[pl.ds(base, SRP)], acc.at[pl.ds(base, SRP)])

    plsc.subcore_barrier()

    def chunk(j, carry):
        pltpu.sync_copy(ones_v, acc.at[col_v.at[j]], add=True)
        return carry

    lax.fori_loop(0, CPW, chunk, 0)

    plsc.subcore_barrier()
    pltpu.sync_copy(acc.at[pl.ds(base, SRP)], out_hbm.at[cid, pl.ds(base, SRP)])


_sc_deg = pl.kernel(
    _sc_deg_body,
    out_type=jax.ShapeDtypeStruct((NC, NP, F), jnp.float32),
    mesh=_MESH,
    scratch_types=[
        pltpu.VMEM((CPW, K), jnp.int32),
        pltpu.VMEM((K, F), jnp.float32),
        pltpu.VMEM_SHARED((NP, F), jnp.float32),
        pltpu.SemaphoreType.DMA,
    ],
)


def _sc_spmm_body(xs_hbm, row_hbm, col_hbm, z_hbm, out_hbm,
                  row_v, col_v, rows_v, acc, sem):
    cid = lax.axis_index("c")
    sid = lax.axis_index("s")
    wid = cid * NS + sid
    base = sid * SRP

    pltpu.sync_copy(row_hbm.at[pl.ds(wid * CPW, CPW)], row_v)
    pltpu.sync_copy(col_hbm.at[pl.ds(wid * CPW, CPW)], col_v)
    pltpu.sync_copy(z_hbm.at[pl.ds(base, SRP)], acc.at[pl.ds(base, SRP)])

    plsc.subcore_barrier()

    def chunk(j, carry):
        pltpu.async_copy(xs_hbm.at[row_v.at[j]], rows_v, sem).wait()
        pltpu.sync_copy(rows_v, acc.at[col_v.at[j]], add=True)
        return carry

    lax.fori_loop(0, CPW, chunk, 0)

    plsc.subcore_barrier()
    pltpu.sync_copy(acc.at[pl.ds(base, SRP)], out_hbm.at[cid, pl.ds(base, SRP)])


_sc_spmm = pl.kernel(
    _sc_spmm_body,
    out_type=jax.ShapeDtypeStruct((NC, NP, F), jnp.float32),
    mesh=_MESH,
    scratch_types=[
        pltpu.VMEM((CPW, K), jnp.int32),
        pltpu.VMEM((CPW, K), jnp.int32),
        pltpu.VMEM((K, F), jnp.float32),
        pltpu.VMEM_SHARED((NP, F), jnp.float32),
        pltpu.SemaphoreType.DMA,
    ],
)


R = 1000
GRID = N // R


def _k1_body(x_ref, w_ref, degp_ref, dinv_ref, xs_ref):
    deg = degp_ref[0][:, 0:1] + degp_ref[1][:, 0:1] + 1.0
    dinv = lax.rsqrt(deg)
    dinv_ref[...] = dinv
    xw = jnp.dot(x_ref[...], w_ref[...], preferred_element_type=jnp.float32)
    xs_ref[...] = xw * dinv


def _k1(x, w, degp):
    return pl.pallas_call(
        _k1_body,
        grid=(GRID,),
        in_specs=[
            pl.BlockSpec((R, F), lambda i: (i, 0)),
            pl.BlockSpec((F, F), lambda i: (0, 0)),
            pl.BlockSpec((NC, R, F), lambda i: (0, i, 0)),
        ],
        out_specs=[
            pl.BlockSpec((R, 1), lambda i: (i, 0)),
            pl.BlockSpec((R, F), lambda i: (i, 0)),
        ],
        out_shape=[
            jax.ShapeDtypeStruct((N, 1), jnp.float32),
            jax.ShapeDtypeStruct((N, F), jnp.float32),
        ],
    )(x, w, degp)


def _k2_body(p_ref, xs_ref, dinv_ref, b_ref, w_ref, out_ref):
    dinv = dinv_ref[...]
    h = jnp.maximum(dinv * (p_ref[0] + p_ref[1] + xs_ref[...]) + b_ref[...], 0.0)
    xw = jnp.dot(h, w_ref[...], preferred_element_type=jnp.float32)
    out_ref[...] = xw * dinv


def _k2(p, xs, dinv, b, w):
    return pl.pallas_call(
        _k2_body,
        grid=(GRID,),
        in_specs=[
            pl.BlockSpec((NC, R, F), lambda i: (0, i, 0)),
            pl.BlockSpec((R, F), lambda i: (i, 0)),
            pl.BlockSpec((R, 1), lambda i: (i, 0)),
            pl.BlockSpec((1, F), lambda i: (0, 0)),
            pl.BlockSpec((F, F), lambda i: (0, 0)),
        ],
        out_specs=pl.BlockSpec((R, F), lambda i: (i, 0)),
        out_shape=jax.ShapeDtypeStruct((N, F), jnp.float32),
    )(p, xs, dinv, b, w)


def _k4_body(p_ref, xs_ref, dinv_ref, b_ref, wl_ref, bl_ref, out_ref):
    x3 = dinv_ref[...] * (p_ref[0] + p_ref[1] + xs_ref[...]) + b_ref[...]
    logits = jnp.dot(x3, wl_ref[...], preferred_element_type=jnp.float32)
    logits = logits + bl_ref[...]
    m = jnp.max(logits, axis=1, keepdims=True)
    s = jnp.sum(jnp.exp(logits - m), axis=1, keepdims=True)
    out_ref[...] = logits - m - jnp.log(s)


def _k4(p, xs, dinv, b, wl, bl):
    return pl.pallas_call(
        _k4_body,
        grid=(GRID,),
        in_specs=[
            pl.BlockSpec((NC, R, F), lambda i: (0, i, 0)),
            pl.BlockSpec((R, F), lambda i: (i, 0)),
            pl.BlockSpec((R, 1), lambda i: (i, 0)),
            pl.BlockSpec((1, F), lambda i: (0, 0)),
            pl.BlockSpec((F, NCLASS), lambda i: (0, 0)),
            pl.BlockSpec((1, NCLASS), lambda i: (0, 0)),
        ],
        out_specs=pl.BlockSpec((R, NCLASS), lambda i: (i, 0)),
        out_shape=jax.ShapeDtypeStruct((N, NCLASS), jnp.float32),
    )(p, xs, dinv, b, wl, bl)


def kernel(x, edge_index, W1, b1, W2, b2, W3, b3, Wl, bl):
    row = edge_index[0]
    col = edge_index[1]
    pad = EPAD - E
    rowp = jnp.concatenate([row, jnp.zeros((pad,), jnp.int32)]).reshape(NW * CPW, K)
    colp = jnp.concatenate([col, jnp.full((pad,), N, jnp.int32)]).reshape(NW * CPW, K)

    z128 = jnp.zeros((NP, F), jnp.float32)
    ones128 = jnp.ones((K, F), jnp.float32)

    degp = _sc_deg(colp, z128, ones128)
    dinv, xs = _k1(x, W1, degp)

    p = _sc_spmm(xs, rowp, colp, z128)
    xs = _k2(p, xs, dinv, b1.reshape(1, F), W2)
    p = _sc_spmm(xs, rowp, colp, z128)
    xs = _k2(p, xs, dinv, b2.reshape(1, F), W3)
    p = _sc_spmm(xs, rowp, colp, z128)
    return _k4(p, xs, dinv, b3.reshape(1, F), Wl, bl.reshape(1, NCLASS))


# speedup vs baseline: 8.2560x; 1.1232x over previous
import jax
import jax.numpy as jnp
from jax import lax
from jax.experimental import pallas as pl
from jax.experimental.pallas import tpu as pltpu
from jax.experimental.pallas import tpu_sc as plsc

N = 10000
E = 320000
F = 128
NCLASS = 40

NC = 2
NS = 16
NW = NC * NS
K = 128
CPW = 80
EPW = CPW * K
EPAD = NW * EPW
SRP = 632
NP = NS * SRP

_MESH = plsc.VectorSubcoreMesh(
    core_axis_name="c", subcore_axis_name="s", num_cores=NC, num_subcores=NS)


def _sc_deg_body(col_hbm, z_hbm, ones_hbm, out_hbm, col_v, ones_v, acc, sem):
    cid = lax.axis_index("c")
    sid = lax.axis_index("s")
    wid = cid * NS + sid
    base = sid * SRP

    pltpu.sync_copy(col_hbm.at[pl.ds(wid * CPW, CPW)], col_v)
    pltpu.sync_copy(ones_hbm, ones_v)
    pltpu.sync_copy(z_hbm.at[pl.ds(base, SRP)], acc.at[pl.ds(base, SRP)])

    plsc.subcore_barrier()

    def chunk(j, carry):
        pltpu.sync_copy(ones_v, acc.at[col_v.at[j]], add=True)
        return carry

    lax.fori_loop(0, CPW, chunk, 0)

    plsc.subcore_barrier()
    pltpu.sync_copy(acc.at[pl.ds(base, SRP)], out_hbm.at[cid, pl.ds(base, SRP)])


_sc_deg = pl.kernel(
    _sc_deg_body,
    out_type=jax.ShapeDtypeStruct((NC, NP, F), jnp.float32),
    mesh=_MESH,
    scratch_types=[
        pltpu.VMEM((CPW, K), jnp.int32),
        pltpu.VMEM((K, F), jnp.float32),
        pltpu.VMEM_SHARED((NP, F), jnp.float32),
        pltpu.SemaphoreType.DMA,
    ],
)


NBUF = 3


def _sc_spmm_body(xs_hbm, rc_hbm, z_hbm, out_hbm,
                  ix0, ix1, ix2, rv0, rv1, rv2, acc,
                  is0, is1, is2, gs0, gs1, gs2):
    cid = lax.axis_index("c")
    sid = lax.axis_index("s")
    wid = cid * NS + sid
    base = sid * SRP
    ixs = (ix0, ix1, ix2)
    rvs = (rv0, rv1, rv2)
    isems = (is0, is1, is2)
    gsems = (gs0, gs1, gs2)
    c0 = wid * CPW

    pltpu.sync_copy(z_hbm.at[pl.ds(base, SRP)], acc.at[pl.ds(base, SRP)])

    plsc.subcore_barrier()

    for b in range(NBUF):
        pltpu.async_copy(rc_hbm.at[c0 + b], ixs[b], isems[b])
    for b in range(2):
        pltpu.make_async_copy(rc_hbm.at[c0 + b], ixs[b], isems[b]).wait()
        pltpu.async_copy(xs_hbm.at[ixs[b].at[0]], rvs[b], gsems[b])

    def group(jj, carry):
        for b in range(NBUF):
            j = jj * NBUF + b
            pltpu.make_async_copy(xs_hbm.at[ixs[b].at[0]], rvs[b], gsems[b]).wait()
            pltpu.sync_copy(rvs[b], acc.at[ixs[b].at[1]], add=True)

            @pl.when(j + NBUF < CPW)
            def _():
                pltpu.async_copy(rc_hbm.at[c0 + j + NBUF], ixs[b], isems[b])

            @pl.when(j + 2 < CPW)
            def _():
                b2 = (b + 2) % NBUF
                pltpu.make_async_copy(
                    rc_hbm.at[c0 + j + 2], ixs[b2], isems[b2]).wait()
                pltpu.async_copy(xs_hbm.at[ixs[b2].at[0]], rvs[b2], gsems[b2])
        return carry

    lax.fori_loop(0, CPW // NBUF, group, 0, unroll=False)

    for j in (CPW - 2, CPW - 1):
        b = j % NBUF
        pltpu.make_async_copy(xs_hbm.at[ixs[b].at[0]], rvs[b], gsems[b]).wait()
        pltpu.sync_copy(rvs[b], acc.at[ixs[b].at[1]], add=True)

    plsc.subcore_barrier()
    pltpu.sync_copy(acc.at[pl.ds(base, SRP)], out_hbm.at[cid, pl.ds(base, SRP)])


_sc_spmm = pl.kernel(
    _sc_spmm_body,
    out_type=jax.ShapeDtypeStruct((NC, NP, F), jnp.float32),
    mesh=_MESH,
    scratch_types=[
        pltpu.VMEM((2, K), jnp.int32),
        pltpu.VMEM((2, K), jnp.int32),
        pltpu.VMEM((2, K), jnp.int32),
        pltpu.VMEM((K, F), jnp.float32),
        pltpu.VMEM((K, F), jnp.float32),
        pltpu.VMEM((K, F), jnp.float32),
        pltpu.VMEM_SHARED((NP, F), jnp.float32),
        pltpu.SemaphoreType.DMA,
        pltpu.SemaphoreType.DMA,
        pltpu.SemaphoreType.DMA,
        pltpu.SemaphoreType.DMA,
        pltpu.SemaphoreType.DMA,
        pltpu.SemaphoreType.DMA,
    ],
)


R = 1000
GRID = N // R


def _k1_body(x_ref, w_ref, degp_ref, dinv_ref, xs_ref):
    deg = degp_ref[0][:, 0:1] + degp_ref[1][:, 0:1] + 1.0
    dinv = lax.rsqrt(deg)
    dinv_ref[...] = dinv
    xw = jnp.dot(x_ref[...], w_ref[...], preferred_element_type=jnp.float32)
    xs_ref[...] = xw * dinv


def _k1(x, w, degp):
    return pl.pallas_call(
        _k1_body,
        grid=(GRID,),
        in_specs=[
            pl.BlockSpec((R, F), lambda i: (i, 0)),
            pl.BlockSpec((F, F), lambda i: (0, 0)),
            pl.BlockSpec((NC, R, F), lambda i: (0, i, 0)),
        ],
        out_specs=[
            pl.BlockSpec((R, 1), lambda i: (i, 0)),
            pl.BlockSpec((R, F), lambda i: (i, 0)),
        ],
        out_shape=[
            jax.ShapeDtypeStruct((N, 1), jnp.float32),
            jax.ShapeDtypeStruct((N, F), jnp.float32),
        ],
    )(x, w, degp)


def _k2_body(p_ref, xs_ref, dinv_ref, b_ref, w_ref, out_ref):
    dinv = dinv_ref[...]
    h = jnp.maximum(dinv * (p_ref[0] + p_ref[1] + xs_ref[...]) + b_ref[...], 0.0)
    xw = jnp.dot(h, w_ref[...], preferred_element_type=jnp.float32)
    out_ref[...] = xw * dinv


def _k2(p, xs, dinv, b, w):
    return pl.pallas_call(
        _k2_body,
        grid=(GRID,),
        in_specs=[
            pl.BlockSpec((NC, R, F), lambda i: (0, i, 0)),
            pl.BlockSpec((R, F), lambda i: (i, 0)),
            pl.BlockSpec((R, 1), lambda i: (i, 0)),
            pl.BlockSpec((1, F), lambda i: (0, 0)),
            pl.BlockSpec((F, F), lambda i: (0, 0)),
        ],
        out_specs=pl.BlockSpec((R, F), lambda i: (i, 0)),
        out_shape=jax.ShapeDtypeStruct((N, F), jnp.float32),
    )(p, xs, dinv, b, w)


def _k4_body(p_ref, xs_ref, dinv_ref, b_ref, wl_ref, bl_ref, out_ref):
    x3 = dinv_ref[...] * (p_ref[0] + p_ref[1] + xs_ref[...]) + b_ref[...]
    logits = jnp.dot(x3, wl_ref[...], preferred_element_type=jnp.float32)
    logits = logits + bl_ref[...]
    m = jnp.max(logits, axis=1, keepdims=True)
    s = jnp.sum(jnp.exp(logits - m), axis=1, keepdims=True)
    out_ref[...] = logits - m - jnp.log(s)


def _k4(p, xs, dinv, b, wl, bl):
    return pl.pallas_call(
        _k4_body,
        grid=(GRID,),
        in_specs=[
            pl.BlockSpec((NC, R, F), lambda i: (0, i, 0)),
            pl.BlockSpec((R, F), lambda i: (i, 0)),
            pl.BlockSpec((R, 1), lambda i: (i, 0)),
            pl.BlockSpec((1, F), lambda i: (0, 0)),
            pl.BlockSpec((F, NCLASS), lambda i: (0, 0)),
            pl.BlockSpec((1, NCLASS), lambda i: (0, 0)),
        ],
        out_specs=pl.BlockSpec((R, NCLASS), lambda i: (i, 0)),
        out_shape=jax.ShapeDtypeStruct((N, NCLASS), jnp.float32),
    )(p, xs, dinv, b, wl, bl)


def kernel(x, edge_index, W1, b1, W2, b2, W3, b3, Wl, bl):
    row = edge_index[0]
    col = edge_index[1]
    pad = EPAD - E
    rowp = jnp.concatenate([row, jnp.zeros((pad,), jnp.int32)]).reshape(NW * CPW, K)
    colp = jnp.concatenate([col, jnp.full((pad,), N, jnp.int32)]).reshape(NW * CPW, K)
    rc = jnp.concatenate(
        [rowp.reshape(NW * CPW, 1, K), colp.reshape(NW * CPW, 1, K)], axis=1)

    z128 = jnp.zeros((NP, F), jnp.float32)
    ones128 = jnp.ones((K, F), jnp.float32)

    degp = _sc_deg(colp, z128, ones128)
    dinv, xs = _k1(x, W1, degp)

    p = _sc_spmm(xs, rc, z128)
    xs = _k2(p, xs, dinv, b1.reshape(1, F), W2)
    p = _sc_spmm(xs, rc, z128)
    xs = _k2(p, xs, dinv, b2.reshape(1, F), W3)
    p = _sc_spmm(xs, rc, z128)
    return _k4(p, xs, dinv, b3.reshape(1, F), Wl, bl.reshape(1, NCLASS))


# speedup vs baseline: 26.1893x; 3.1721x over previous
import jax
import jax.numpy as jnp
from jax import lax
from jax.experimental import pallas as pl
from jax.experimental.pallas import tpu as pltpu
from jax.experimental.pallas import tpu_sc as plsc

N = 10000
E = 320000
F = 128
NCLASS = 40

NC = 2
NS = 16
NW = NC * NS
K = 64
CPW = 160
EPW = CPW * K
EPAD = NW * EPW
SRP = 632
NP = NS * SRP

_MESH = plsc.VectorSubcoreMesh(
    core_axis_name="c", subcore_axis_name="s", num_cores=NC, num_subcores=NS)


def _sc_deg_body(col_hbm, z_hbm, ones_hbm, out_hbm, col_v, ones_v, acc, sem):
    cid = lax.axis_index("c")
    sid = lax.axis_index("s")
    wid = cid * NS + sid
    base = sid * SRP

    pltpu.sync_copy(col_hbm.at[pl.ds(wid * CPW, CPW)], col_v)
    pltpu.sync_copy(ones_hbm, ones_v)
    pltpu.sync_copy(z_hbm.at[pl.ds(base, SRP)], acc.at[pl.ds(base, SRP)])

    plsc.subcore_barrier()

    def chunk(j, carry):
        pltpu.sync_copy(ones_v, acc.at[col_v.at[j]], add=True)
        return carry

    lax.fori_loop(0, CPW, chunk, 0)

    plsc.subcore_barrier()
    pltpu.sync_copy(acc.at[pl.ds(base, SRP)], out_hbm.at[cid, pl.ds(base, SRP)])


_sc_deg = pl.kernel(
    _sc_deg_body,
    out_type=jax.ShapeDtypeStruct((NC, NP, F), jnp.float32),
    mesh=_MESH,
    scratch_types=[
        pltpu.VMEM((CPW, K), jnp.int32),
        pltpu.VMEM((K, F), jnp.float32),
        pltpu.VMEM_SHARED((NP, F), jnp.float32),
        pltpu.SemaphoreType.DMA,
    ],
)


NSLOT = 5


def _sc_spmm_body(xs_hbm, rc_hbm, z_hbm, out_hbm, *rest):
    ixs = rest[:NSLOT]
    rvs = rest[NSLOT:2 * NSLOT]
    acc = rest[2 * NSLOT]
    isems = rest[2 * NSLOT + 1:3 * NSLOT + 1]
    gsems = rest[3 * NSLOT + 1:]
    cid = lax.axis_index("c")
    sid = lax.axis_index("s")
    wid = cid * NS + sid
    base = sid * SRP
    c0 = wid * CPW

    pltpu.sync_copy(z_hbm.at[pl.ds(base, SRP)], acc.at[pl.ds(base, SRP)])

    plsc.subcore_barrier()

    for b in range(NSLOT):
        pltpu.async_copy(rc_hbm.at[c0 + b], ixs[b], isems[b])
    for b in range(NSLOT - 1):
        pltpu.make_async_copy(rc_hbm.at[c0 + b], ixs[b], isems[b]).wait()
        pltpu.async_copy(xs_hbm.at[ixs[b].at[0]], rvs[b], gsems[b])

    def group(jj, carry):
        for b in range(NSLOT):
            j = jj * NSLOT + b
            pltpu.make_async_copy(xs_hbm.at[ixs[b].at[0]], rvs[b], gsems[b]).wait()
            pltpu.sync_copy(rvs[b], acc.at[ixs[b].at[1]], add=True)

            @pl.when(j + NSLOT < CPW)
            def _():
                pltpu.async_copy(rc_hbm.at[c0 + j + NSLOT], ixs[b], isems[b])

            @pl.when(j + NSLOT - 1 < CPW)
            def _():
                b2 = (b + NSLOT - 1) % NSLOT
                pltpu.make_async_copy(
                    rc_hbm.at[c0 + j + NSLOT - 1], ixs[b2], isems[b2]).wait()
                pltpu.async_copy(xs_hbm.at[ixs[b2].at[0]], rvs[b2], gsems[b2])
        return carry

    lax.fori_loop(0, CPW // NSLOT, group, 0, unroll=False)

    for j in range(CPW - CPW % NSLOT, CPW):
        b = j % NSLOT
        pltpu.make_async_copy(xs_hbm.at[ixs[b].at[0]], rvs[b], gsems[b]).wait()
        pltpu.sync_copy(rvs[b], acc.at[ixs[b].at[1]], add=True)

    plsc.subcore_barrier()
    pltpu.sync_copy(acc.at[pl.ds(base, SRP)], out_hbm.at[cid, pl.ds(base, SRP)])


_sc_spmm = pl.kernel(
    _sc_spmm_body,
    out_type=jax.ShapeDtypeStruct((NC, NP, F), jnp.float32),
    mesh=_MESH,
    scratch_types=(
        [pltpu.VMEM((2, K), jnp.int32) for _ in range(NSLOT)]
        + [pltpu.VMEM((K, F), jnp.float32) for _ in range(NSLOT)]
        + [pltpu.VMEM_SHARED((NP, F), jnp.float32)]
        + [pltpu.SemaphoreType.DMA for _ in range(2 * NSLOT)]
    ),
)


R = 1000
GRID = N // R


def _k1_body(x_ref, w_ref, degp_ref, dinv_ref, xs_ref):
    deg = degp_ref[0][:, 0:1] + degp_ref[1][:, 0:1] + 1.0
    dinv = lax.rsqrt(deg)
    dinv_ref[...] = dinv
    xw = jnp.dot(x_ref[...], w_ref[...], preferred_element_type=jnp.float32)
    xs_ref[...] = xw * dinv


def _k1(x, w, degp):
    return pl.pallas_call(
        _k1_body,
        grid=(GRID,),
        in_specs=[
            pl.BlockSpec((R, F), lambda i: (i, 0)),
            pl.BlockSpec((F, F), lambda i: (0, 0)),
            pl.BlockSpec((NC, R, F), lambda i: (0, i, 0)),
        ],
        out_specs=[
            pl.BlockSpec((R, 1), lambda i: (i, 0)),
            pl.BlockSpec((R, F), lambda i: (i, 0)),
        ],
        out_shape=[
            jax.ShapeDtypeStruct((N, 1), jnp.float32),
            jax.ShapeDtypeStruct((N, F), jnp.float32),
        ],
    )(x, w, degp)


def _k2_body(p_ref, xs_ref, dinv_ref, b_ref, w_ref, out_ref):
    dinv = dinv_ref[...]
    h = jnp.maximum(dinv * (p_ref[0] + p_ref[1] + xs_ref[...]) + b_ref[...], 0.0)
    xw = jnp.dot(h, w_ref[...], preferred_element_type=jnp.float32)
    out_ref[...] = xw * dinv


def _k2(p, xs, dinv, b, w):
    return pl.pallas_call(
        _k2_body,
        grid=(GRID,),
        in_specs=[
            pl.BlockSpec((NC, R, F), lambda i: (0, i, 0)),
            pl.BlockSpec((R, F), lambda i: (i, 0)),
            pl.BlockSpec((R, 1), lambda i: (i, 0)),
            pl.BlockSpec((1, F), lambda i: (0, 0)),
            pl.BlockSpec((F, F), lambda i: (0, 0)),
        ],
        out_specs=pl.BlockSpec((R, F), lambda i: (i, 0)),
        out_shape=jax.ShapeDtypeStruct((N, F), jnp.float32),
    )(p, xs, dinv, b, w)


def _k4_body(p_ref, xs_ref, dinv_ref, b_ref, wl_ref, bl_ref, out_ref):
    x3 = dinv_ref[...] * (p_ref[0] + p_ref[1] + xs_ref[...]) + b_ref[...]
    logits = jnp.dot(x3, wl_ref[...], preferred_element_type=jnp.float32)
    logits = logits + bl_ref[...]
    m = jnp.max(logits, axis=1, keepdims=True)
    s = jnp.sum(jnp.exp(logits - m), axis=1, keepdims=True)
    out_ref[...] = logits - m - jnp.log(s)


def _k4(p, xs, dinv, b, wl, bl):
    return pl.pallas_call(
        _k4_body,
        grid=(GRID,),
        in_specs=[
            pl.BlockSpec((NC, R, F), lambda i: (0, i, 0)),
            pl.BlockSpec((R, F), lambda i: (i, 0)),
            pl.BlockSpec((R, 1), lambda i: (i, 0)),
            pl.BlockSpec((1, F), lambda i: (0, 0)),
            pl.BlockSpec((F, NCLASS), lambda i: (0, 0)),
            pl.BlockSpec((1, NCLASS), lambda i: (0, 0)),
        ],
        out_specs=pl.BlockSpec((R, NCLASS), lambda i: (i, 0)),
        out_shape=jax.ShapeDtypeStruct((N, NCLASS), jnp.float32),
    )(p, xs, dinv, b, wl, bl)


def kernel(x, edge_index, W1, b1, W2, b2, W3, b3, Wl, bl):
    row = edge_index[0]
    col = edge_index[1]
    pad = EPAD - E
    prow = (jnp.arange(pad, dtype=jnp.int32) * 37) % N
    pcol = N + (jnp.arange(pad, dtype=jnp.int32) % (NP - N))
    rowp = jnp.concatenate([row, prow]).reshape(NW * CPW, K)
    colp = jnp.concatenate([col, pcol]).reshape(NW * CPW, K)
    rc = jnp.concatenate(
        [rowp.reshape(NW * CPW, 1, K), colp.reshape(NW * CPW, 1, K)], axis=1)

    z128 = jnp.zeros((NP, F), jnp.float32)
    ones128 = jnp.ones((K, F), jnp.float32)

    degp = _sc_deg(colp, z128, ones128)
    dinv, xs = _k1(x, W1, degp)

    p = _sc_spmm(xs, rc, z128)
    xs = _k2(p, xs, dinv, b1.reshape(1, F), W2)
    p = _sc_spmm(xs, rc, z128)
    xs = _k2(p, xs, dinv, b2.reshape(1, F), W3)
    p = _sc_spmm(xs, rc, z128)
    return _k4(p, xs, dinv, b3.reshape(1, F), Wl, bl.reshape(1, NCLASS))
